# Initial kernel scaffold; baseline (speedup 1.0000x reference)
import jax
import jax.numpy as jnp
from jax.experimental import pallas as pl


def kernel(pixel_values, We, Wl, Wb, G1, b1, G2, b2):
    raise NotImplementedError("write your pallas kernel here")


# speedup vs baseline: 1.2166x; 1.2166x over previous
import functools

import jax
import jax.numpy as jnp
from jax.experimental import pallas as pl
from jax.experimental.pallas import tpu as pltpu

B = 8
H = 384
PATCH = 16
NP_PATCHES = (H // PATCH) ** 2
PD = 3 * PATCH * PATCH
D = 768
NC = 92
E = 4
HID = 16
TOPK = 2
NCB = NC + 4


def _moe_body(p_ref, we_ref, wc_ref, g1_ref, b1_ref, g2_ref, b2_ref,
              comb_ref, small_ref, idx_ref, combo_scr):
    b = pl.program_id(0)
    x = p_ref[0]

    for e in range(E):
        f = jnp.maximum(
            jnp.dot(x, we_ref[e], preferred_element_type=jnp.float32), 0.0)
        combo = jnp.dot(f, wc_ref[e], preferred_element_type=jnp.float32)
        boxes = jax.nn.sigmoid(combo[:, NC:NCB])
        combo_scr[e] = jnp.concatenate([combo[:, :NC], boxes], axis=1)

    col0 = combo_scr[:, :, 0]
    ep = jax.nn.sigmoid(
        jnp.sum(col0, axis=1, keepdims=True) / NP_PATCHES).T

    hg = jnp.maximum(
        jnp.dot(ep, g1_ref[...], preferred_element_type=jnp.float32)
        + b1_ref[...], 0.0)
    gl = (jnp.dot(hg, g2_ref[...], preferred_element_type=jnp.float32)
          + b2_ref[...])
    gl = gl - jnp.max(gl, axis=1, keepdims=True)
    eg = jnp.exp(gl)
    w = eg / jnp.sum(eg, axis=1, keepdims=True)

    iota = jax.lax.broadcasted_iota(jnp.int32, (1, E), 1)
    w1 = jnp.max(w, axis=1, keepdims=True)
    i1 = jnp.min(jnp.where(w == w1, iota, E), axis=1, keepdims=True)
    wm = jnp.where(iota == i1, -jnp.inf, w)
    w2 = jnp.max(wm, axis=1, keepdims=True)
    i2 = jnp.min(jnp.where(wm == w2, iota, E), axis=1, keepdims=True)
    keep = (iota == i1) | (iota == i2)
    masked = jnp.where(keep, w, 0.0)
    nw = masked / (jnp.sum(masked, axis=1, keepdims=True) + 1e-8)

    acc = jnp.zeros((NP_PATCHES, NCB), dtype=jnp.float32)
    for e in range(E):
        acc = acc + nw[0, e] * combo_scr[e]
    comb_ref[0] = acc

    fp = jnp.sum(nw * ep, axis=1, keepdims=True)
    row = jnp.concatenate([nw, ep, fp, jnp.zeros((1, 7), jnp.float32)], axis=1)
    small_ref[pl.ds(b, 1), :] = row
    irow = jnp.concatenate(
        [i1, i2, jnp.zeros((1, 6), jnp.int32)], axis=1)
    idx_ref[pl.ds(b, 1), :] = irow


@functools.partial(jax.jit, static_argnames=())
def kernel(pixel_values, We, Wl, Wb, G1, b1, G2, b2):
    nb = H // PATCH
    patches = pixel_values.reshape(B, 3, nb, PATCH, nb, PATCH)
    patches = patches.transpose(0, 2, 4, 1, 3, 5).reshape(B, NP_PATCHES, PD)
    Wc = jnp.concatenate([Wl, Wb], axis=-1)

    comb, small, idx = pl.pallas_call(
        _moe_body,
        grid=(B,),
        in_specs=[
            pl.BlockSpec((1, NP_PATCHES, PD), lambda b: (b, 0, 0)),
            pl.BlockSpec((E, PD, D), lambda b: (0, 0, 0)),
            pl.BlockSpec((E, D, NCB), lambda b: (0, 0, 0)),
            pl.BlockSpec((E, HID), lambda b: (0, 0)),
            pl.BlockSpec((1, HID), lambda b: (0, 0)),
            pl.BlockSpec((HID, E), lambda b: (0, 0)),
            pl.BlockSpec((1, E), lambda b: (0, 0)),
        ],
        out_specs=[
            pl.BlockSpec((1, NP_PATCHES, NCB), lambda b: (b, 0, 0)),
            pl.BlockSpec((B, 16), lambda b: (0, 0)),
            pl.BlockSpec((B, 8), lambda b: (0, 0)),
        ],
        out_shape=[
            jax.ShapeDtypeStruct((B, NP_PATCHES, NCB), jnp.float32),
            jax.ShapeDtypeStruct((B, 16), jnp.float32),
            jax.ShapeDtypeStruct((B, 8), jnp.int32),
        ],
        scratch_shapes=[pltpu.VMEM((E, NP_PATCHES, NCB), jnp.float32)],
    )(patches, We, Wc, G1, b1.reshape(1, HID), G2, b2.reshape(1, E))

    combined_logits = comb[:, :, :NC]
    combined_boxes = comb[:, :, NC:NCB]
    nw = small[:, 0:E]
    expert_probs = small[:, E:2 * E]
    final_pred = small[:, 2 * E]
    top_indices = idx[:, :TOPK]
    return (combined_logits, combined_boxes, final_pred, nw,
            expert_probs, top_indices)


# speedup vs baseline: 1.9078x; 1.5681x over previous
import functools

import jax
import jax.numpy as jnp
from jax.experimental import pallas as pl
from jax.experimental.pallas import tpu as pltpu

B = 8
H = 384
PATCH = 16
NP_PATCHES = (H // PATCH) ** 2
PD = 3 * PATCH * PATCH
D = 768
NC = 92
E = 4
HID = 16
TOPK = 2
NCB = NC + 4


def _moe_body(p_ref, we_ref, wc_ref, g1_ref, b1_ref, g2_ref, b2_ref,
              comb_ref, small_ref, idx_ref, combo_scr):
    b = pl.program_id(0)
    nb = H // PATCH
    pix = p_ref[0]
    x = jnp.transpose(pix, (1, 3, 0, 2, 4)).reshape(NP_PATCHES, PD)

    for e in range(E):
        f = jnp.maximum(
            jnp.dot(x, we_ref[e], preferred_element_type=jnp.float32), 0.0)
        combo = jnp.dot(f, wc_ref[e], preferred_element_type=jnp.float32)
        boxes = jax.nn.sigmoid(combo[:, NC:NCB])
        combo_scr[e] = jnp.concatenate([combo[:, :NC], boxes], axis=1)

    col0 = combo_scr[:, :, 0]
    ep = jax.nn.sigmoid(
        jnp.sum(col0, axis=1, keepdims=True) / NP_PATCHES).T

    hg = jnp.maximum(
        jnp.dot(ep, g1_ref[...], preferred_element_type=jnp.float32)
        + b1_ref[...], 0.0)
    gl = (jnp.dot(hg, g2_ref[...], preferred_element_type=jnp.float32)
          + b2_ref[...])
    gl = gl - jnp.max(gl, axis=1, keepdims=True)
    eg = jnp.exp(gl)
    w = eg / jnp.sum(eg, axis=1, keepdims=True)

    iota = jax.lax.broadcasted_iota(jnp.int32, (1, E), 1)
    w1 = jnp.max(w, axis=1, keepdims=True)
    i1 = jnp.min(jnp.where(w == w1, iota, E), axis=1, keepdims=True)
    wm = jnp.where(iota == i1, -jnp.inf, w)
    w2 = jnp.max(wm, axis=1, keepdims=True)
    i2 = jnp.min(jnp.where(wm == w2, iota, E), axis=1, keepdims=True)
    keep = (iota == i1) | (iota == i2)
    masked = jnp.where(keep, w, 0.0)
    nw = masked / (jnp.sum(masked, axis=1, keepdims=True) + 1e-8)

    acc = jnp.zeros((NP_PATCHES, NCB), dtype=jnp.float32)
    for e in range(E):
        acc = acc + nw[0, e] * combo_scr[e]
    comb_ref[0] = acc

    fp = jnp.sum(nw * ep, axis=1, keepdims=True)
    row = jnp.concatenate([nw, ep, fp, jnp.zeros((1, 7), jnp.float32)], axis=1)
    small_ref[pl.ds(b, 1), :] = row
    irow = jnp.concatenate(
        [i1, i2, jnp.zeros((1, 6), jnp.int32)], axis=1)
    idx_ref[pl.ds(b, 1), :] = irow


@functools.partial(jax.jit, static_argnames=())
def kernel(pixel_values, We, Wl, Wb, G1, b1, G2, b2):
    nb = H // PATCH
    pix = pixel_values.reshape(B, 3, nb, PATCH, nb, PATCH)
    Wc = jnp.concatenate([Wl, Wb], axis=-1)

    comb, small, idx = pl.pallas_call(
        _moe_body,
        grid=(B,),
        in_specs=[
            pl.BlockSpec((1, 3, nb, PATCH, nb, PATCH),
                         lambda b: (b, 0, 0, 0, 0, 0)),
            pl.BlockSpec((E, PD, D), lambda b: (0, 0, 0)),
            pl.BlockSpec((E, D, NCB), lambda b: (0, 0, 0)),
            pl.BlockSpec((E, HID), lambda b: (0, 0)),
            pl.BlockSpec((1, HID), lambda b: (0, 0)),
            pl.BlockSpec((HID, E), lambda b: (0, 0)),
            pl.BlockSpec((1, E), lambda b: (0, 0)),
        ],
        out_specs=[
            pl.BlockSpec((1, NP_PATCHES, NCB), lambda b: (b, 0, 0)),
            pl.BlockSpec((B, 16), lambda b: (0, 0)),
            pl.BlockSpec((B, 8), lambda b: (0, 0)),
        ],
        out_shape=[
            jax.ShapeDtypeStruct((B, NP_PATCHES, NCB), jnp.float32),
            jax.ShapeDtypeStruct((B, 16), jnp.float32),
            jax.ShapeDtypeStruct((B, 8), jnp.int32),
        ],
        scratch_shapes=[pltpu.VMEM((E, NP_PATCHES, NCB), jnp.float32)],
    )(pix, We, Wc, G1, b1.reshape(1, HID), G2, b2.reshape(1, E))

    combined_logits = comb[:, :, :NC]
    combined_boxes = comb[:, :, NC:NCB]
    nw = small[:, 0:E]
    expert_probs = small[:, E:2 * E]
    final_pred = small[:, 2 * E]
    top_indices = idx[:, :TOPK]
    return (combined_logits, combined_boxes, final_pred, nw,
            expert_probs, top_indices)


# speedup vs baseline: 2.6199x; 1.3733x over previous
import functools

import jax
import jax.numpy as jnp
from jax.experimental import pallas as pl
from jax.experimental.pallas import tpu as pltpu

B = 8
H = 384
PATCH = 16
NP_PATCHES = (H // PATCH) ** 2
PD = 3 * PATCH * PATCH
D = 768
NC = 92
E = 4
HID = 16
TOPK = 2
NCB = NC + 4


def _moe_body(p_ref, we_ref, wc_ref, g1_ref, b1_ref, g2_ref, b2_ref,
              comb_ref, small_ref, idx_ref, combo_scr):
    b = pl.program_id(0)
    nb = H // PATCH
    pix = p_ref[0].reshape(3, nb, PATCH, nb, PATCH)
    x = jnp.transpose(pix, (1, 3, 0, 2, 4)).reshape(NP_PATCHES, PD)

    for e in range(E):
        f = jnp.maximum(
            jnp.dot(x, we_ref[e], preferred_element_type=jnp.float32), 0.0)
        combo = jnp.dot(f, wc_ref[e], preferred_element_type=jnp.float32)
        boxes = jax.nn.sigmoid(combo[:, NC:NCB])
        combo_scr[e] = jnp.concatenate([combo[:, :NC], boxes], axis=1)

    col0 = combo_scr[:, :, 0]
    ep = jax.nn.sigmoid(
        jnp.sum(col0, axis=1, keepdims=True) / NP_PATCHES).T

    hg = jnp.maximum(
        jnp.dot(ep, g1_ref[...], preferred_element_type=jnp.float32)
        + b1_ref[...], 0.0)
    gl = (jnp.dot(hg, g2_ref[...], preferred_element_type=jnp.float32)
          + b2_ref[...])
    gl = gl - jnp.max(gl, axis=1, keepdims=True)
    eg = jnp.exp(gl)
    w = eg / jnp.sum(eg, axis=1, keepdims=True)

    iota = jax.lax.broadcasted_iota(jnp.int32, (1, E), 1)
    w1 = jnp.max(w, axis=1, keepdims=True)
    i1 = jnp.min(jnp.where(w == w1, iota, E), axis=1, keepdims=True)
    wm = jnp.where(iota == i1, -jnp.inf, w)
    w2 = jnp.max(wm, axis=1, keepdims=True)
    i2 = jnp.min(jnp.where(wm == w2, iota, E), axis=1, keepdims=True)
    keep = (iota == i1) | (iota == i2)
    masked = jnp.where(keep, w, 0.0)
    nw = masked / (jnp.sum(masked, axis=1, keepdims=True) + 1e-8)

    acc = jnp.zeros((NP_PATCHES, NCB), dtype=jnp.float32)
    for e in range(E):
        acc = acc + nw[0, e] * combo_scr[e]
    comb_ref[0] = acc

    fp = jnp.sum(nw * ep, axis=1, keepdims=True)
    row = jnp.concatenate([nw, ep, fp, jnp.zeros((1, 7), jnp.float32)], axis=1)
    small_ref[pl.ds(b, 1), :] = row
    irow = jnp.concatenate(
        [i1, i2, jnp.zeros((1, 6), jnp.int32)], axis=1)
    idx_ref[pl.ds(b, 1), :] = irow


@functools.partial(jax.jit, static_argnames=())
def kernel(pixel_values, We, Wl, Wb, G1, b1, G2, b2):
    nb = H // PATCH
    Wc = jnp.concatenate([Wl, Wb], axis=-1)

    comb, small, idx = pl.pallas_call(
        _moe_body,
        grid=(B,),
        in_specs=[
            pl.BlockSpec((1, 3, H, H), lambda b: (b, 0, 0, 0)),
            pl.BlockSpec((E, PD, D), lambda b: (0, 0, 0)),
            pl.BlockSpec((E, D, NCB), lambda b: (0, 0, 0)),
            pl.BlockSpec((E, HID), lambda b: (0, 0)),
            pl.BlockSpec((1, HID), lambda b: (0, 0)),
            pl.BlockSpec((HID, E), lambda b: (0, 0)),
            pl.BlockSpec((1, E), lambda b: (0, 0)),
        ],
        out_specs=[
            pl.BlockSpec((1, NP_PATCHES, NCB), lambda b: (b, 0, 0)),
            pl.BlockSpec((B, 16), lambda b: (0, 0)),
            pl.BlockSpec((B, 8), lambda b: (0, 0)),
        ],
        out_shape=[
            jax.ShapeDtypeStruct((B, NP_PATCHES, NCB), jnp.float32),
            jax.ShapeDtypeStruct((B, 16), jnp.float32),
            jax.ShapeDtypeStruct((B, 8), jnp.int32),
        ],
        scratch_shapes=[pltpu.VMEM((E, NP_PATCHES, NCB), jnp.float32)],
    )(pixel_values, We, Wc, G1, b1.reshape(1, HID), G2, b2.reshape(1, E))

    combined_logits = comb[:, :, :NC]
    combined_boxes = comb[:, :, NC:NCB]
    nw = small[:, 0:E]
    expert_probs = small[:, E:2 * E]
    final_pred = small[:, 2 * E]
    top_indices = idx[:, :TOPK]
    return (combined_logits, combined_boxes, final_pred, nw,
            expert_probs, top_indices)


# speedup vs baseline: 2.6838x; 1.0244x over previous
import functools

import jax
import jax.numpy as jnp
from jax.experimental import pallas as pl
from jax.experimental.pallas import tpu as pltpu

B = 8
H = 384
PATCH = 16
NP_PATCHES = (H // PATCH) ** 2
PD = 3 * PATCH * PATCH
D = 768
NC = 92
E = 4
HID = 16
TOPK = 2
NCB = NC + 4


def _moe_body(p_ref, we_ref, wc_ref, g1_ref, b1_ref, g2_ref, b2_ref,
              comb_ref, small_ref, idx_ref, combo_scr):
    b = pl.program_id(0)
    nb = H // PATCH
    T = 4
    ph_per = nb // T
    rows_per = ph_per * nb
    for t in range(T):
        pix_t = p_ref[0, :, t * ph_per * PATCH:(t + 1) * ph_per * PATCH, :]
        pix_t = pix_t.reshape(3, ph_per, PATCH, nb, PATCH)
        xt = jnp.transpose(pix_t, (1, 3, 0, 2, 4)).reshape(rows_per, PD)
        for e in range(E):
            f = jnp.maximum(
                jnp.dot(xt, we_ref[e], preferred_element_type=jnp.float32),
                0.0)
            combo = jnp.dot(f, wc_ref[e], preferred_element_type=jnp.float32)
            boxes = jax.nn.sigmoid(combo[:, NC:NCB])
            combo_scr[e, pl.ds(t * rows_per, rows_per), :] = jnp.concatenate(
                [combo[:, :NC], boxes], axis=1)

    col0 = combo_scr[:, :, 0]
    ep = jax.nn.sigmoid(
        jnp.sum(col0, axis=1, keepdims=True) / NP_PATCHES).T

    hg = jnp.maximum(
        jnp.dot(ep, g1_ref[...], preferred_element_type=jnp.float32)
        + b1_ref[...], 0.0)
    gl = (jnp.dot(hg, g2_ref[...], preferred_element_type=jnp.float32)
          + b2_ref[...])
    gl = gl - jnp.max(gl, axis=1, keepdims=True)
    eg = jnp.exp(gl)
    w = eg / jnp.sum(eg, axis=1, keepdims=True)

    iota = jax.lax.broadcasted_iota(jnp.int32, (1, E), 1)
    w1 = jnp.max(w, axis=1, keepdims=True)
    i1 = jnp.min(jnp.where(w == w1, iota, E), axis=1, keepdims=True)
    wm = jnp.where(iota == i1, -jnp.inf, w)
    w2 = jnp.max(wm, axis=1, keepdims=True)
    i2 = jnp.min(jnp.where(wm == w2, iota, E), axis=1, keepdims=True)
    keep = (iota == i1) | (iota == i2)
    masked = jnp.where(keep, w, 0.0)
    nw = masked / (jnp.sum(masked, axis=1, keepdims=True) + 1e-8)

    acc = jnp.zeros((NP_PATCHES, NCB), dtype=jnp.float32)
    for e in range(E):
        acc = acc + nw[0, e] * combo_scr[e]
    comb_ref[0] = acc

    fp = jnp.sum(nw * ep, axis=1, keepdims=True)
    row = jnp.concatenate([nw, ep, fp, jnp.zeros((1, 7), jnp.float32)], axis=1)
    small_ref[pl.ds(b, 1), :] = row
    irow = jnp.concatenate(
        [i1, i2, jnp.zeros((1, 6), jnp.int32)], axis=1)
    idx_ref[pl.ds(b, 1), :] = irow


@functools.partial(jax.jit, static_argnames=())
def kernel(pixel_values, We, Wl, Wb, G1, b1, G2, b2):
    nb = H // PATCH
    Wc = jnp.concatenate([Wl, Wb], axis=-1)

    comb, small, idx = pl.pallas_call(
        _moe_body,
        grid=(B,),
        in_specs=[
            pl.BlockSpec((1, 3, H, H), lambda b: (b, 0, 0, 0)),
            pl.BlockSpec((E, PD, D), lambda b: (0, 0, 0)),
            pl.BlockSpec((E, D, NCB), lambda b: (0, 0, 0)),
            pl.BlockSpec((E, HID), lambda b: (0, 0)),
            pl.BlockSpec((1, HID), lambda b: (0, 0)),
            pl.BlockSpec((HID, E), lambda b: (0, 0)),
            pl.BlockSpec((1, E), lambda b: (0, 0)),
        ],
        out_specs=[
            pl.BlockSpec((1, NP_PATCHES, NCB), lambda b: (b, 0, 0)),
            pl.BlockSpec((B, 16), lambda b: (0, 0)),
            pl.BlockSpec((B, 8), lambda b: (0, 0)),
        ],
        out_shape=[
            jax.ShapeDtypeStruct((B, NP_PATCHES, NCB), jnp.float32),
            jax.ShapeDtypeStruct((B, 16), jnp.float32),
            jax.ShapeDtypeStruct((B, 8), jnp.int32),
        ],
        scratch_shapes=[pltpu.VMEM((E, NP_PATCHES, NCB), jnp.float32)],
    )(pixel_values, We, Wc, G1, b1.reshape(1, HID), G2, b2.reshape(1, E))

    combined_logits = comb[:, :, :NC]
    combined_boxes = comb[:, :, NC:NCB]
    nw = small[:, 0:E]
    expert_probs = small[:, E:2 * E]
    final_pred = small[:, 2 * E]
    top_indices = idx[:, :TOPK]
    return (combined_logits, combined_boxes, final_pred, nw,
            expert_probs, top_indices)


# speedup vs baseline: 3.0695x; 1.1437x over previous
import functools

import jax
import jax.numpy as jnp
from jax.experimental import pallas as pl
from jax.experimental.pallas import tpu as pltpu

B = 8
H = 384
PATCH = 16
NP_PATCHES = (H // PATCH) ** 2
PD = 3 * PATCH * PATCH
D = 768
NC = 92
E = 4
HID = 16
TOPK = 2
NCB = NC + 4


def _moe_body(p_ref, we_ref, wc_ref, g1_ref, b1_ref, g2_ref, b2_ref,
              comb_ref, small_ref, idx_ref, combo_scr):
    b = pl.program_id(0)
    nb = H // PATCH
    T = 4
    ph_per = nb // T
    rows_per = ph_per * nb
    for t in range(T):
        pix_t = p_ref[0, :, t * ph_per * PATCH:(t + 1) * ph_per * PATCH, :]
        pix_t = pix_t.reshape(3, ph_per, PATCH, nb, PATCH)
        pix_t = pix_t.astype(jnp.bfloat16)
        xt = jnp.transpose(pix_t, (1, 3, 0, 2, 4)).reshape(rows_per, PD)
        for e in range(E):
            f = jnp.maximum(
                jnp.dot(xt, we_ref[e], preferred_element_type=jnp.float32),
                0.0).astype(jnp.bfloat16)
            combo = jnp.dot(f, wc_ref[e], preferred_element_type=jnp.float32)
            boxes = jax.nn.sigmoid(combo[:, NC:NCB])
            combo_scr[e, pl.ds(t * rows_per, rows_per), :] = jnp.concatenate(
                [combo[:, :NC], boxes], axis=1)

    col0 = combo_scr[:, :, 0]
    ep = jax.nn.sigmoid(
        jnp.sum(col0, axis=1, keepdims=True) / NP_PATCHES).T

    hg = jnp.maximum(
        jnp.dot(ep, g1_ref[...], preferred_element_type=jnp.float32)
        + b1_ref[...], 0.0)
    gl = (jnp.dot(hg, g2_ref[...], preferred_element_type=jnp.float32)
          + b2_ref[...])
    gl = gl - jnp.max(gl, axis=1, keepdims=True)
    eg = jnp.exp(gl)
    w = eg / jnp.sum(eg, axis=1, keepdims=True)

    iota = jax.lax.broadcasted_iota(jnp.int32, (1, E), 1)
    w1 = jnp.max(w, axis=1, keepdims=True)
    i1 = jnp.min(jnp.where(w == w1, iota, E), axis=1, keepdims=True)
    wm = jnp.where(iota == i1, -jnp.inf, w)
    w2 = jnp.max(wm, axis=1, keepdims=True)
    i2 = jnp.min(jnp.where(wm == w2, iota, E), axis=1, keepdims=True)
    keep = (iota == i1) | (iota == i2)
    masked = jnp.where(keep, w, 0.0)
    nw = masked / (jnp.sum(masked, axis=1, keepdims=True) + 1e-8)

    acc = jnp.zeros((NP_PATCHES, NCB), dtype=jnp.float32)
    for e in range(E):
        acc = acc + nw[0, e] * combo_scr[e]
    comb_ref[0] = acc

    fp = jnp.sum(nw * ep, axis=1, keepdims=True)
    row = jnp.concatenate([nw, ep, fp, jnp.zeros((1, 7), jnp.float32)], axis=1)
    small_ref[pl.ds(b, 1), :] = row
    irow = jnp.concatenate(
        [i1, i2, jnp.zeros((1, 6), jnp.int32)], axis=1)
    idx_ref[pl.ds(b, 1), :] = irow


@functools.partial(jax.jit, static_argnames=())
def kernel(pixel_values, We, Wl, Wb, G1, b1, G2, b2):
    nb = H // PATCH
    Wc = jnp.concatenate([Wl, Wb], axis=-1).astype(jnp.bfloat16)
    We16 = We.astype(jnp.bfloat16)

    comb, small, idx = pl.pallas_call(
        _moe_body,
        grid=(B,),
        in_specs=[
            pl.BlockSpec((1, 3, H, H), lambda b: (b, 0, 0, 0)),
            pl.BlockSpec((E, PD, D), lambda b: (0, 0, 0)),
            pl.BlockSpec((E, D, NCB), lambda b: (0, 0, 0)),
            pl.BlockSpec((E, HID), lambda b: (0, 0)),
            pl.BlockSpec((1, HID), lambda b: (0, 0)),
            pl.BlockSpec((HID, E), lambda b: (0, 0)),
            pl.BlockSpec((1, E), lambda b: (0, 0)),
        ],
        out_specs=[
            pl.BlockSpec((1, NP_PATCHES, NCB), lambda b: (b, 0, 0)),
            pl.BlockSpec((B, 16), lambda b: (0, 0)),
            pl.BlockSpec((B, 8), lambda b: (0, 0)),
        ],
        out_shape=[
            jax.ShapeDtypeStruct((B, NP_PATCHES, NCB), jnp.float32),
            jax.ShapeDtypeStruct((B, 16), jnp.float32),
            jax.ShapeDtypeStruct((B, 8), jnp.int32),
        ],
        scratch_shapes=[pltpu.VMEM((E, NP_PATCHES, NCB), jnp.float32)],
    )(pixel_values, We16, Wc, G1, b1.reshape(1, HID), G2, b2.reshape(1, E))

    combined_logits = comb[:, :, :NC]
    combined_boxes = comb[:, :, NC:NCB]
    nw = small[:, 0:E]
    expert_probs = small[:, E:2 * E]
    final_pred = small[:, 2 * E]
    top_indices = idx[:, :TOPK]
    return (combined_logits, combined_boxes, final_pred, nw,
            expert_probs, top_indices)
